# baseline (speedup 1.0000x reference)
import jax
import jax.numpy as jnp
from jax import lax
from jax.experimental import pallas as pl
from jax.experimental.pallas import tpu as pltpu
from jax.experimental.pallas import tpu_sc as plsc

F = 81920
B = 1024
M = 4
F_SC = 8192
F_TC = F - F_SC
BF = 2048
NSTEPS = F_TC // BF

NC, NS = 2, 16
NW = NC * NS
ROWS = B // NW
CH = 512
R_BLK = 8
NCH = F_SC // CH


def _tc_main(wf_ref, bf_ref, ftwT_ref, wout_ref, bout_ref, wacc, bacc):
    i = pl.program_id(0)

    @pl.when(i == 0)
    def _init():
        wacc[...] = jnp.zeros_like(wacc)
        bacc[...] = jnp.zeros_like(bacc)

    ftwT = ftwT_ref[...]
    wacc[...] += jnp.dot(wf_ref[...], ftwT, preferred_element_type=jnp.float32)
    bacc[...] += jnp.dot(bf_ref[...], ftwT, preferred_element_type=jnp.float32)

    @pl.when(i == NSTEPS - 1)
    def _out():
        wout_ref[...] = wacc[...]
        bout_ref[...] = bacc[...]


def _sc_body(wf_hbm, bf_hbm, ftw_hbm, wout_hbm, bout_hbm,
             fw0, fw1, fb0, fb1, wt, accw, accb,
             sw0, sw1, sb0, sb1, swt):
    wid = lax.axis_index("s") * NC + lax.axis_index("c")
    row0 = wid * ROWS

    wt_copy = pltpu.async_copy(ftw_hbm.at[:, pl.ds(F_TC, F_SC)], wt, swt)

    fw = (fw0, fw1)
    fb = (fb0, fb1)
    sw = (sw0, sw1)
    sb = (sb0, sb1)

    def start(c, slot):
        def _issue():
            f0 = F_TC + c * CH
            pltpu.async_copy(
                wf_hbm.at[pl.ds(row0, ROWS), pl.ds(f0, CH)], fw[slot], sw[slot])
            pltpu.async_copy(
                bf_hbm.at[pl.ds(row0, ROWS), pl.ds(f0, CH)], fb[slot], sb[slot])

        if isinstance(c, int):
            if c < NCH:
                _issue()
        else:
            pl.when(c < NCH)(_issue)

    def wait(slot):
        pltpu.make_async_copy(
            wf_hbm.at[pl.ds(row0, ROWS), pl.ds(0, CH)], fw[slot],
            sw[slot]).wait()
        pltpu.make_async_copy(
            bf_hbm.at[pl.ds(row0, ROWS), pl.ds(0, CH)], fb[slot],
            sb[slot]).wait()

    zero = jnp.zeros((16,), jnp.float32)
    for r in range(ROWS):
        for m in range(M):
            accw[r, m, :] = zero
            accb[r, m, :] = zero

    start(0, 0)
    start(1, 1)
    wt_copy.wait()

    def compute_chunk(c, slot):
        woff = c * CH
        for fbuf, acc in ((fw[slot], accw), (fb[slot], accb)):
            for rg in range(ROWS // R_BLK):
                r0 = rg * R_BLK

                def jbody(j, accs, fbuf=fbuf, r0=r0, woff=woff):
                    ws = tuple(wt[m, pl.ds(woff + j * 16, 16)]
                               for m in range(M))
                    out = []
                    for r in range(R_BLK):
                        fv = fbuf[r0 + r, pl.ds(j * 16, 16)]
                        row = accs[r]
                        out.append(tuple(row[m] + fv * ws[m]
                                         for m in range(M)))
                    return tuple(out)

                accs0 = tuple(tuple(zero for _ in range(M))
                              for _ in range(R_BLK))
                accs = lax.fori_loop(0, CH // 16, jbody, accs0)
                for r in range(R_BLK):
                    for m in range(M):
                        plsc.addupdate(acc.at[r0 + r, m], accs[r][m])

    def pair_body(g, _):
        c0 = g * 2
        for b in range(2):
            wait(b)
            compute_chunk(c0 + b, b)
            start(c0 + b + 2, b)
        return _

    lax.fori_loop(0, NCH // 2, pair_body, 0)

    pltpu.sync_copy(accw, wout_hbm.at[pl.ds(row0, ROWS)])
    pltpu.sync_copy(accb, bout_hbm.at[pl.ds(row0, ROWS)])


def _sc_partial(white_features, black_features, ft_w):
    mesh = plsc.VectorSubcoreMesh(core_axis_name="c", subcore_axis_name="s",
                                  num_cores=NC, num_subcores=NS)
    return pl.kernel(
        _sc_body,
        out_type=[jax.ShapeDtypeStruct((B, M, 16), jnp.float32),
                  jax.ShapeDtypeStruct((B, M, 16), jnp.float32)],
        mesh=mesh,
        scratch_types=[
            pltpu.VMEM((ROWS, CH), jnp.float32),
            pltpu.VMEM((ROWS, CH), jnp.float32),
            pltpu.VMEM((ROWS, CH), jnp.float32),
            pltpu.VMEM((ROWS, CH), jnp.float32),
            pltpu.VMEM((M, F_SC), jnp.float32),
            pltpu.VMEM((ROWS, M, 16), jnp.float32),
            pltpu.VMEM((ROWS, M, 16), jnp.float32),
            pltpu.SemaphoreType.DMA,
            pltpu.SemaphoreType.DMA,
            pltpu.SemaphoreType.DMA,
            pltpu.SemaphoreType.DMA,
            pltpu.SemaphoreType.DMA,
        ],
    )(white_features, black_features, ft_w)


def _epilogue(wtc_ref, btc_ref, wsc_ref, bsc_ref, ftb_ref, turn_ref,
              score_ref, result_ref, l1wT_ref, l1b_ref, l2wT_ref, l2b_ref,
              out_ref):
    sel = (lax.broadcasted_iota(jnp.int32, (4 * 16, M), 0) // 16
           == lax.broadcasted_iota(jnp.int32, (4 * 16, M), 1)
           ).astype(jnp.float32)
    ftb = ftb_ref[...]
    w = (wtc_ref[...]
         + jnp.dot(wsc_ref[...], sel, preferred_element_type=jnp.float32)
         + ftb)
    b = (btc_ref[...]
         + jnp.dot(bsc_ref[...], sel, preferred_element_type=jnp.float32)
         + ftb)
    turn = turn_ref[...]
    acc_wb = jnp.concatenate([w, b], axis=1)
    acc_bw = jnp.concatenate([b, w], axis=1)
    accumulator = turn * acc_wb + (1.0 - turn) * acc_bw
    l1_x = jnp.clip(accumulator, 0.0, 1.0)
    l2_in = jnp.dot(l1_x, l1wT_ref[...],
                    preferred_element_type=jnp.float32) + l1b_ref[...]
    l2_x = jnp.clip(l2_in, 0.0, 1.0)
    model_result = jnp.dot(l2_x, l2wT_ref[...],
                           preferred_element_type=jnp.float32) + l2b_ref[...]
    wdl_m = jax.nn.sigmoid(model_result / 400.0)
    wdl_t = jax.nn.sigmoid(score_ref[...] / 400.0)
    loss = 0.5 * (wdl_m - wdl_t) ** 2 + 0.5 * (wdl_m - result_ref[...]) ** 2
    out_ref[...] = loss


def kernel(white_features, black_features, turn, score, result,
           ft_w, ft_b, l1_w, l1_b, l2_w, l2_b):
    wtc, btc = pl.pallas_call(
        _tc_main,
        grid=(NSTEPS,),
        in_specs=[
            pl.BlockSpec((B, BF), lambda i: (0, i)),
            pl.BlockSpec((B, BF), lambda i: (0, i)),
            pl.BlockSpec((BF, M), lambda i: (i, 0)),
        ],
        out_specs=[pl.BlockSpec((B, M), lambda i: (0, 0)),
                   pl.BlockSpec((B, M), lambda i: (0, 0))],
        out_shape=[jax.ShapeDtypeStruct((B, M), jnp.float32),
                   jax.ShapeDtypeStruct((B, M), jnp.float32)],
        scratch_shapes=[pltpu.VMEM((B, M), jnp.float32),
                        pltpu.VMEM((B, M), jnp.float32)],
    )(white_features, black_features, ft_w.T)

    wsc, bsc = _sc_partial(white_features, black_features, ft_w)

    return pl.pallas_call(
        _epilogue,
        out_shape=jax.ShapeDtypeStruct((B, 1), jnp.float32),
    )(wtc, btc, wsc.reshape(B, M * 16), bsc.reshape(B, M * 16),
      ft_b.reshape(1, M), turn, score, result,
      l1_w.T, l1_b.reshape(1, 8), l2_w.T, l2_b.reshape(1, 1))
